# baseline (speedup 1.0000x reference)
import jax
import jax.numpy as jnp
from jax.experimental import pallas as pl
from jax.experimental.pallas import tpu as pltpu


def kernel(x, table):
    B, S, H = x.shape
    CH = 512
    NT = S // CH
    NCHUNK = NT * B
    NBUF = 4

    def body(x_hbm, t_hbm, o_hbm, b0, b1, b2, b3, t0, t1,
             si0, si1, si2, si3, so0, so1, so2, so3, st0, st1):
        bufs = (b0, b1, b2, b3)
        tbufs = (t0, t1)
        sin = (si0, si1, si2, si3)
        sout = (so0, so1, so2, so3)
        stin = (st0, st1)

        def row0(c):
            si, b = divmod(c, B)
            return b * S + si * CH

        def fire_in(c):
            k = c % NBUF
            return pltpu.async_copy(
                x_hbm.at[pl.ds(row0(c), CH), :], bufs[k], sin[k])

        def fire_tin(si):
            k = si % 2
            return pltpu.async_copy(
                t_hbm.at[pl.ds(si * CH, CH), :], tbufs[k], stin[k])

        t_h = [None] * NT
        in_h = [None] * NCHUNK
        out_h = [None] * NCHUNK
        t_h[0] = fire_tin(0)
        for c in range(NBUF - 1):
            in_h[c] = fire_in(c)
        for c in range(NCHUNK):
            k = c % NBUF
            si, b = divmod(c, B)
            if c == 1 and NT > 1:
                t_h[1] = fire_tin(1)
            if c + NBUF - 1 < NCHUNK:
                if c >= 1:
                    out_h[c - 1].wait()
                in_h[c + NBUF - 1] = fire_in(c + NBUF - 1)
            in_h[c].wait()
            if b == 0:
                t_h[si].wait()
            buf = bufs[k]
            buf[...] = buf[...] + tbufs[si % 2][...]
            out_h[c] = pltpu.async_copy(
                buf, o_hbm.at[pl.ds(row0(c), CH), :], sout[k])
            if b == B - 1 and si + 2 < NT:
                t_h[si + 2] = fire_tin(si + 2)
        for c in range(NCHUNK - NBUF, NCHUNK):
            out_h[c].wait()

    R = B * S
    out = pl.pallas_call(
        body,
        in_specs=[
            pl.BlockSpec(memory_space=pltpu.MemorySpace.HBM),
            pl.BlockSpec(memory_space=pltpu.MemorySpace.HBM),
        ],
        out_specs=pl.BlockSpec(memory_space=pltpu.MemorySpace.HBM),
        out_shape=jax.ShapeDtypeStruct((R, H), jnp.float32),
        scratch_shapes=(
            [pltpu.VMEM((CH, H), jnp.float32)] * (NBUF + 2)
            + [pltpu.SemaphoreType.DMA] * (NBUF * 2 + 2)
        ),
    )(x.reshape(R, H), table[:S])
    return out.reshape(B, S, H)


# speedup vs baseline: 1.2596x; 1.2596x over previous
import jax
import jax.numpy as jnp
from jax.experimental import pallas as pl
from jax.experimental.pallas import tpu as pltpu


def kernel(x, table):
    B, S, H = x.shape
    CH = 512
    NT = S // CH
    NCHUNK = NT * B
    NBUF = 8
    AH = 4

    def body(x_hbm, t_hbm, o_hbm, *rest):
        bufs = rest[:NBUF]
        tbufs = rest[NBUF:NBUF + 2]
        sin = rest[NBUF + 2:2 * NBUF + 2]
        sout = rest[2 * NBUF + 2:3 * NBUF + 2]
        stin = rest[3 * NBUF + 2:]

        def row0(c):
            si, b = divmod(c, B)
            return b * S + si * CH

        def fire_in(c):
            k = c % NBUF
            return pltpu.async_copy(
                x_hbm.at[pl.ds(row0(c), CH), :], bufs[k], sin[k])

        def fire_tin(si):
            k = si % 2
            return pltpu.async_copy(
                t_hbm.at[pl.ds(si * CH, CH), :], tbufs[k], stin[k])

        t_h = [None] * NT
        in_h = [None] * NCHUNK
        out_h = [None] * NCHUNK
        waited = set()
        t_h[0] = fire_tin(0)
        for c in range(AH):
            in_h[c] = fire_in(c)
        for c in range(NCHUNK):
            k = c % NBUF
            si, b = divmod(c, B)
            if c == 1 and NT > 1:
                t_h[1] = fire_tin(1)
            if c + AH < NCHUNK:
                j = c + AH - NBUF
                if j >= 0:
                    out_h[j].wait()
                    waited.add(j)
                in_h[c + AH] = fire_in(c + AH)
            in_h[c].wait()
            if b == 0:
                t_h[si].wait()
            buf = bufs[k]
            buf[...] = buf[...] + tbufs[si % 2][...]
            out_h[c] = pltpu.async_copy(
                buf, o_hbm.at[pl.ds(row0(c), CH), :], sout[k])
            if b == B - 1 and si + 2 < NT:
                t_h[si + 2] = fire_tin(si + 2)
        for c in range(NCHUNK):
            if c not in waited:
                out_h[c].wait()

    R = B * S
    out = pl.pallas_call(
        body,
        in_specs=[
            pl.BlockSpec(memory_space=pltpu.MemorySpace.HBM),
            pl.BlockSpec(memory_space=pltpu.MemorySpace.HBM),
        ],
        out_specs=pl.BlockSpec(memory_space=pltpu.MemorySpace.HBM),
        out_shape=jax.ShapeDtypeStruct((R, H), jnp.float32),
        scratch_shapes=(
            [pltpu.VMEM((CH, H), jnp.float32)] * (NBUF + 2)
            + [pltpu.SemaphoreType.DMA] * (NBUF * 2 + 2)
        ),
        compiler_params=pltpu.CompilerParams(
            vmem_limit_bytes=100 * 1024 * 1024),
    )(x.reshape(R, H), table[:S])
    return out.reshape(B, S, H)


# speedup vs baseline: 1.3499x; 1.0717x over previous
import jax
import jax.numpy as jnp
from jax.experimental import pallas as pl


def _add_kernel(x_ref, t_ref, o_ref):
    o_ref[...] = x_ref[...] + t_ref[...][None, :, :]


def kernel(x, table):
    B, S, H = x.shape
    BB = 2
    grid = (B // BB,)
    return pl.pallas_call(
        _add_kernel,
        grid=grid,
        in_specs=[
            pl.BlockSpec((BB, S, H), lambda j: (j, 0, 0)),
            pl.BlockSpec((S, H), lambda j: (0, 0)),
        ],
        out_specs=pl.BlockSpec((BB, S, H), lambda j: (j, 0, 0)),
        out_shape=jax.ShapeDtypeStruct((B, S, H), x.dtype),
    )(x, table[:S])
